# baseline (speedup 1.0000x reference)
import jax
import jax.numpy as jnp
from jax.experimental import pallas as pl
from jax.experimental.pallas import tpu as pltpu

B = 4
N = 4096
TILE_R = 2048
NR = N // TILE_R
K = 15
NC = 4
CW = N // NC
BIG = 1e10


def _split2(x):
    h = x.astype(jnp.bfloat16)
    l = (x - h.astype(jnp.float32)).astype(jnp.bfloat16)
    return h, l


def _split3(x):
    h = x.astype(jnp.bfloat16)
    r = x - h.astype(jnp.float32)
    m = r.astype(jnp.bfloat16)
    l = (r - m.astype(jnp.float32)).astype(jnp.bfloat16)
    return h, m, l


def _chamfer_body(pr_ref, gt_ref, val_ref, out_ref, ga_ref, colmin_ref, fwd_ref):
    b = pl.program_id(0)
    r = pl.program_id(1)
    nr = pl.num_programs(1)

    @pl.when(jnp.logical_and(b == 0, r == 0))
    def _init_out():
        out_ref[0, 0] = jnp.float32(0.0)

    @pl.when(r == 0)
    def _init_batch():
        colmin_ref[...] = jnp.full(colmin_ref.shape, BIG, jnp.float32)
        fwd_ref[0, 0] = jnp.float32(0.0)
        gt = gt_ref[0]
        val = val_ref[0]
        gn = jnp.sum(gt * gt, axis=0, keepdims=True)
        gnp = gn + (jnp.float32(1.0) - val) * jnp.float32(BIG)
        gh, gl = _split2(gt)
        gnh, gnm, gnl = _split3(gnp)
        ga_ref[...] = jnp.concatenate(
            [gh, gl, gh,
             jnp.ones((3, gt.shape[1]), jnp.bfloat16),
             gnh, gnm, gnl], axis=0)

    pr = pr_ref[0]
    pn = jnp.sum(pr * pr, axis=0, keepdims=True)
    p2h, p2l = _split2(pr * jnp.float32(-2.0))
    pnh, pnm, pnl = _split3(pn)
    pa = jnp.concatenate(
        [p2h, p2h, p2l,
         pnh, pnm, pnl,
         jnp.ones((3, pr.shape[1]), jnp.bfloat16)], axis=0)

    rm = None
    for c in range(NC):
        sl = pl.ds(c * CW, CW)
        dm_c = jax.lax.dot_general(pa, ga_ref[:, sl],
                                   (((0,), (0,)), ((), ())),
                                   preferred_element_type=jnp.float32)
        rm_c = jnp.min(dm_c, axis=1, keepdims=True)
        rm = rm_c if rm is None else jnp.minimum(rm, rm_c)
        colmin_ref[:, sl] = jnp.minimum(colmin_ref[:, sl],
                                        jnp.min(dm_c, axis=0, keepdims=True))

    fwd_ref[0, 0] += jnp.sum(rm)

    @pl.when(r == nr - 1)
    def _finish_batch():
        val = val_ref[0]
        lf = fwd_ref[0, 0] / jnp.float32(N)
        vsum = jnp.sum(val)
        lb = jnp.sum(colmin_ref[...] * val) / jnp.maximum(vsum, 1.0)
        out_ref[0, 0] += (jnp.float32(2.0) / B) * (lf + lb)


def kernel(pts3d_xyz, target_pts3d, target_valid):
    pr_t = jnp.swapaxes(pts3d_xyz, 1, 2)
    gt_t = jnp.swapaxes(target_pts3d, 1, 2)
    val_f = target_valid.astype(jnp.float32)[:, None, :]

    out = pl.pallas_call(
        _chamfer_body,
        grid=(B, NR),
        in_specs=[
            pl.BlockSpec((1, 3, TILE_R), lambda b, r: (b, 0, r)),
            pl.BlockSpec((1, 3, N), lambda b, r: (b, 0, 0)),
            pl.BlockSpec((1, 1, N), lambda b, r: (b, 0, 0)),
        ],
        out_specs=pl.BlockSpec(memory_space=pltpu.SMEM),
        out_shape=jax.ShapeDtypeStruct((1, 1), jnp.float32),
        scratch_shapes=[
            pltpu.VMEM((K, N), jnp.bfloat16),
            pltpu.VMEM((1, N), jnp.float32),
            pltpu.SMEM((1, 1), jnp.float32),
        ],
        compiler_params=pltpu.CompilerParams(
            dimension_semantics=("arbitrary", "arbitrary"),
        ),
    )(pr_t, gt_t, val_f)
    return out[0, 0]


# speedup vs baseline: 1.0662x; 1.0662x over previous
import jax
import jax.numpy as jnp
from jax.experimental import pallas as pl
from jax.experimental.pallas import tpu as pltpu

B = 4
N = 4096
TILE_R = 4096
NR = N // TILE_R
K = 15
NC = 4
CW = N // NC
BIG = 1e10


def _split2(x):
    h = x.astype(jnp.bfloat16)
    l = (x - h.astype(jnp.float32)).astype(jnp.bfloat16)
    return h, l


def _split3(x):
    h = x.astype(jnp.bfloat16)
    r = x - h.astype(jnp.float32)
    m = r.astype(jnp.bfloat16)
    l = (r - m.astype(jnp.float32)).astype(jnp.bfloat16)
    return h, m, l


def _chamfer_body(pr_ref, gt_ref, val_ref, out_ref, ga_ref, colmin_ref, fwd_ref):
    b = pl.program_id(0)
    r = pl.program_id(1)
    nr = pl.num_programs(1)

    @pl.when(jnp.logical_and(b == 0, r == 0))
    def _init_out():
        out_ref[0, 0] = jnp.float32(0.0)

    @pl.when(r == 0)
    def _init_batch():
        colmin_ref[...] = jnp.full(colmin_ref.shape, BIG, jnp.float32)
        fwd_ref[0, 0] = jnp.float32(0.0)
        gt = gt_ref[0]
        val = val_ref[0]
        gn = jnp.sum(gt * gt, axis=0, keepdims=True)
        gnp = gn + (jnp.float32(1.0) - val) * jnp.float32(BIG)
        gh, gl = _split2(gt)
        gnh, gnm, gnl = _split3(gnp)
        ga_ref[...] = jnp.concatenate(
            [gh, gl, gh,
             jnp.ones((3, gt.shape[1]), jnp.bfloat16),
             gnh, gnm, gnl], axis=0)

    pr = pr_ref[0]
    pn = jnp.sum(pr * pr, axis=0, keepdims=True)
    p2h, p2l = _split2(pr * jnp.float32(-2.0))
    pnh, pnm, pnl = _split3(pn)
    pa = jnp.concatenate(
        [p2h, p2h, p2l,
         pnh, pnm, pnl,
         jnp.ones((3, pr.shape[1]), jnp.bfloat16)], axis=0)

    rm = None
    for c in range(NC):
        sl = pl.ds(c * CW, CW)
        dm_c = jax.lax.dot_general(pa, ga_ref[:, sl],
                                   (((0,), (0,)), ((), ())),
                                   preferred_element_type=jnp.float32)
        rm_c = jnp.min(dm_c, axis=1, keepdims=True)
        rm = rm_c if rm is None else jnp.minimum(rm, rm_c)
        colmin_ref[:, sl] = jnp.minimum(colmin_ref[:, sl],
                                        jnp.min(dm_c, axis=0, keepdims=True))

    fwd_ref[0, 0] += jnp.sum(rm)

    @pl.when(r == nr - 1)
    def _finish_batch():
        val = val_ref[0]
        lf = fwd_ref[0, 0] / jnp.float32(N)
        vsum = jnp.sum(val)
        lb = jnp.sum(colmin_ref[...] * val) / jnp.maximum(vsum, 1.0)
        out_ref[0, 0] += (jnp.float32(2.0) / B) * (lf + lb)


def kernel(pts3d_xyz, target_pts3d, target_valid):
    pr_t = jnp.swapaxes(pts3d_xyz, 1, 2)
    gt_t = jnp.swapaxes(target_pts3d, 1, 2)
    val_f = target_valid.astype(jnp.float32)[:, None, :]

    out = pl.pallas_call(
        _chamfer_body,
        grid=(B, NR),
        in_specs=[
            pl.BlockSpec((1, 3, TILE_R), lambda b, r: (b, 0, r)),
            pl.BlockSpec((1, 3, N), lambda b, r: (b, 0, 0)),
            pl.BlockSpec((1, 1, N), lambda b, r: (b, 0, 0)),
        ],
        out_specs=pl.BlockSpec(memory_space=pltpu.SMEM),
        out_shape=jax.ShapeDtypeStruct((1, 1), jnp.float32),
        scratch_shapes=[
            pltpu.VMEM((K, N), jnp.bfloat16),
            pltpu.VMEM((1, N), jnp.float32),
            pltpu.SMEM((1, 1), jnp.float32),
        ],
        compiler_params=pltpu.CompilerParams(
            dimension_semantics=("arbitrary", "arbitrary"),
        ),
    )(pr_t, gt_t, val_f)
    return out[0, 0]


# speedup vs baseline: 1.0724x; 1.0059x over previous
import jax
import jax.numpy as jnp
from jax.experimental import pallas as pl
from jax.experimental.pallas import tpu as pltpu

B = 4
N = 4096
TILE_R = 4096
NR = N // TILE_R
K = 15
NC = 8
CW = N // NC
BIG = 1e10


def _split2(x):
    h = x.astype(jnp.bfloat16)
    l = (x - h.astype(jnp.float32)).astype(jnp.bfloat16)
    return h, l


def _split3(x):
    h = x.astype(jnp.bfloat16)
    r = x - h.astype(jnp.float32)
    m = r.astype(jnp.bfloat16)
    l = (r - m.astype(jnp.float32)).astype(jnp.bfloat16)
    return h, m, l


def _chamfer_body(pr_ref, gt_ref, val_ref, out_ref, ga_ref, colmin_ref, fwd_ref):
    b = pl.program_id(0)
    r = pl.program_id(1)
    nr = pl.num_programs(1)

    @pl.when(jnp.logical_and(b == 0, r == 0))
    def _init_out():
        out_ref[0, 0] = jnp.float32(0.0)

    @pl.when(r == 0)
    def _init_batch():
        colmin_ref[...] = jnp.full(colmin_ref.shape, BIG, jnp.float32)
        fwd_ref[0, 0] = jnp.float32(0.0)
        gt = gt_ref[0]
        val = val_ref[0]
        gn = jnp.sum(gt * gt, axis=0, keepdims=True)
        gnp = gn + (jnp.float32(1.0) - val) * jnp.float32(BIG)
        gh, gl = _split2(gt)
        gnh, gnm, gnl = _split3(gnp)
        ga_ref[...] = jnp.concatenate(
            [gh, gl, gh,
             jnp.ones((3, gt.shape[1]), jnp.bfloat16),
             gnh, gnm, gnl], axis=0)

    pr = pr_ref[0]
    pn = jnp.sum(pr * pr, axis=0, keepdims=True)
    p2h, p2l = _split2(pr * jnp.float32(-2.0))
    pnh, pnm, pnl = _split3(pn)
    pa = jnp.concatenate(
        [p2h, p2h, p2l,
         pnh, pnm, pnl,
         jnp.ones((3, pr.shape[1]), jnp.bfloat16)], axis=0)

    rm = None
    for c in range(NC):
        sl = pl.ds(c * CW, CW)
        dm_c = jax.lax.dot_general(pa, ga_ref[:, sl],
                                   (((0,), (0,)), ((), ())),
                                   preferred_element_type=jnp.float32)
        rm_c = jnp.min(dm_c, axis=1, keepdims=True)
        rm = rm_c if rm is None else jnp.minimum(rm, rm_c)
        colmin_ref[:, sl] = jnp.minimum(colmin_ref[:, sl],
                                        jnp.min(dm_c, axis=0, keepdims=True))

    fwd_ref[0, 0] += jnp.sum(rm)

    @pl.when(r == nr - 1)
    def _finish_batch():
        val = val_ref[0]
        lf = fwd_ref[0, 0] / jnp.float32(N)
        vsum = jnp.sum(val)
        lb = jnp.sum(colmin_ref[...] * val) / jnp.maximum(vsum, 1.0)
        out_ref[0, 0] += (jnp.float32(2.0) / B) * (lf + lb)


def kernel(pts3d_xyz, target_pts3d, target_valid):
    pr_t = jnp.swapaxes(pts3d_xyz, 1, 2)
    gt_t = jnp.swapaxes(target_pts3d, 1, 2)
    val_f = target_valid.astype(jnp.float32)[:, None, :]

    out = pl.pallas_call(
        _chamfer_body,
        grid=(B, NR),
        in_specs=[
            pl.BlockSpec((1, 3, TILE_R), lambda b, r: (b, 0, r)),
            pl.BlockSpec((1, 3, N), lambda b, r: (b, 0, 0)),
            pl.BlockSpec((1, 1, N), lambda b, r: (b, 0, 0)),
        ],
        out_specs=pl.BlockSpec(memory_space=pltpu.SMEM),
        out_shape=jax.ShapeDtypeStruct((1, 1), jnp.float32),
        scratch_shapes=[
            pltpu.VMEM((K, N), jnp.bfloat16),
            pltpu.VMEM((1, N), jnp.float32),
            pltpu.SMEM((1, 1), jnp.float32),
        ],
        compiler_params=pltpu.CompilerParams(
            dimension_semantics=("arbitrary", "arbitrary"),
        ),
    )(pr_t, gt_t, val_f)
    return out[0, 0]


# speedup vs baseline: 1.0914x; 1.0177x over previous
import jax
import jax.numpy as jnp
from jax.experimental import pallas as pl
from jax.experimental.pallas import tpu as pltpu

B = 4
N = 4096
TILE_R = 4096
NR = N // TILE_R
K = 15
NC = 2
CW = N // NC
BIG = 1e10


def _split2(x):
    h = x.astype(jnp.bfloat16)
    l = (x - h.astype(jnp.float32)).astype(jnp.bfloat16)
    return h, l


def _split3(x):
    h = x.astype(jnp.bfloat16)
    r = x - h.astype(jnp.float32)
    m = r.astype(jnp.bfloat16)
    l = (r - m.astype(jnp.float32)).astype(jnp.bfloat16)
    return h, m, l


def _chamfer_body(pr_ref, gt_ref, val_ref, out_ref, ga_ref, colmin_ref, fwd_ref):
    b = pl.program_id(0)
    r = pl.program_id(1)
    nr = pl.num_programs(1)

    @pl.when(jnp.logical_and(b == 0, r == 0))
    def _init_out():
        out_ref[0, 0] = jnp.float32(0.0)

    @pl.when(r == 0)
    def _init_batch():
        colmin_ref[...] = jnp.full(colmin_ref.shape, BIG, jnp.float32)
        fwd_ref[0, 0] = jnp.float32(0.0)
        gt = gt_ref[0]
        val = val_ref[0]
        gn = jnp.sum(gt * gt, axis=0, keepdims=True)
        gnp = gn + (jnp.float32(1.0) - val) * jnp.float32(BIG)
        gh, gl = _split2(gt)
        gnh, gnm, gnl = _split3(gnp)
        ga_ref[...] = jnp.concatenate(
            [gh, gl, gh,
             jnp.ones((3, gt.shape[1]), jnp.bfloat16),
             gnh, gnm, gnl], axis=0)

    pr = pr_ref[0]
    pn = jnp.sum(pr * pr, axis=0, keepdims=True)
    p2h, p2l = _split2(pr * jnp.float32(-2.0))
    pnh, pnm, pnl = _split3(pn)
    pa = jnp.concatenate(
        [p2h, p2h, p2l,
         pnh, pnm, pnl,
         jnp.ones((3, pr.shape[1]), jnp.bfloat16)], axis=0)

    rm = None
    for c in range(NC):
        sl = pl.ds(c * CW, CW)
        dm_c = jax.lax.dot_general(pa, ga_ref[:, sl],
                                   (((0,), (0,)), ((), ())),
                                   preferred_element_type=jnp.float32)
        rm_c = jnp.min(dm_c, axis=1, keepdims=True)
        rm = rm_c if rm is None else jnp.minimum(rm, rm_c)
        colmin_ref[:, sl] = jnp.minimum(colmin_ref[:, sl],
                                        jnp.min(dm_c, axis=0, keepdims=True))

    fwd_ref[0, 0] += jnp.sum(rm)

    @pl.when(r == nr - 1)
    def _finish_batch():
        val = val_ref[0]
        lf = fwd_ref[0, 0] / jnp.float32(N)
        vsum = jnp.sum(val)
        lb = jnp.sum(colmin_ref[...] * val) / jnp.maximum(vsum, 1.0)
        out_ref[0, 0] += (jnp.float32(2.0) / B) * (lf + lb)


def kernel(pts3d_xyz, target_pts3d, target_valid):
    pr_t = jnp.swapaxes(pts3d_xyz, 1, 2)
    gt_t = jnp.swapaxes(target_pts3d, 1, 2)
    val_f = target_valid.astype(jnp.float32)[:, None, :]

    out = pl.pallas_call(
        _chamfer_body,
        grid=(B, NR),
        in_specs=[
            pl.BlockSpec((1, 3, TILE_R), lambda b, r: (b, 0, r)),
            pl.BlockSpec((1, 3, N), lambda b, r: (b, 0, 0)),
            pl.BlockSpec((1, 1, N), lambda b, r: (b, 0, 0)),
        ],
        out_specs=pl.BlockSpec(memory_space=pltpu.SMEM),
        out_shape=jax.ShapeDtypeStruct((1, 1), jnp.float32),
        scratch_shapes=[
            pltpu.VMEM((K, N), jnp.bfloat16),
            pltpu.VMEM((1, N), jnp.float32),
            pltpu.SMEM((1, 1), jnp.float32),
        ],
        compiler_params=pltpu.CompilerParams(
            dimension_semantics=("arbitrary", "arbitrary"),
        ),
    )(pr_t, gt_t, val_f)
    return out[0, 0]
